# baseline (speedup 1.0000x reference)
import functools

import jax
import jax.numpy as jnp
from jax import lax
from jax.experimental import pallas as pl
from jax.experimental.pallas import tpu as pltpu
from jax.experimental.pallas import tpu_sc as plsc

N = 10000
E = 160000
E_PAD = 163840
IDX_ROWS = E_PAD // 128
NUM_FEAT = 128
DIM = 32
G = 64
NACC = 10016
NC, NS = 2, 16
NW = NC * NS
ROWS_PW = IDX_ROWS // NW
RPT = NACC // NS
BE = 640
F32 = jnp.float32

_MESH = plsc.VectorSubcoreMesh(
    core_axis_name="c", subcore_axis_name="s", num_cores=NC, num_subcores=NS)
_SC_PARAMS = pltpu.CompilerParams(use_tc_tiling_on_sc=False)


def _sc_gather(table, idx2d):
    @functools.partial(
        pl.kernel, mesh=_MESH, compiler_params=_SC_PARAMS,
        out_type=jax.ShapeDtypeStruct((E_PAD, DIM), F32),
        scratch_types=[
            pltpu.VMEM((ROWS_PW, 128), jnp.int32),
            pltpu.VMEM((128, DIM), F32),
            pltpu.SemaphoreType.DMA,
        ],
    )
    def k(table_hbm, idx_hbm, out_hbm, idx_v, buf_v, sem):
        c = lax.axis_index("c")
        s = lax.axis_index("s")
        wid = s * NC + c
        base = wid * ROWS_PW
        pltpu.sync_copy(idx_hbm.at[pl.ds(base, ROWS_PW)], idx_v)

        def body(j, carry):
            pltpu.async_copy(table_hbm.at[idx_v.at[j]], buf_v, sem).wait()
            pltpu.sync_copy(buf_v, out_hbm.at[pl.ds((base + j) * 128, 128)])
            return carry

        lax.fori_loop(0, ROWS_PW, body, 0)

    return k(table, idx2d)


def _sc_scatter(vals, idx2d, zinit):
    @functools.partial(
        pl.kernel, mesh=_MESH, compiler_params=_SC_PARAMS,
        out_type=jax.ShapeDtypeStruct((NC * NACC, DIM), F32),
        scratch_types=[
            pltpu.VMEM((ROWS_PW, 128), jnp.int32),
            pltpu.VMEM((128, DIM), F32),
            pltpu.VMEM_SHARED((NACC, DIM), F32),
            pltpu.SemaphoreType.DMA,
        ],
    )
    def k(val_hbm, idx_hbm, z_hbm, out_hbm, idx_v, buf_v, acc_sh, sem):
        c = lax.axis_index("c")
        s = lax.axis_index("s")
        wid = s * NC + c
        pltpu.sync_copy(z_hbm.at[pl.ds(s * RPT, RPT)],
                        acc_sh.at[pl.ds(s * RPT, RPT)])
        plsc.subcore_barrier()
        base = wid * ROWS_PW
        pltpu.sync_copy(idx_hbm.at[pl.ds(base, ROWS_PW)], idx_v)

        def body(j, carry):
            pltpu.sync_copy(val_hbm.at[pl.ds((base + j) * 128, 128)], buf_v)
            pltpu.sync_copy(buf_v, acc_sh.at[idx_v.at[j]], add=True)
            return carry

        lax.fori_loop(0, ROWS_PW, body, 0)
        plsc.subcore_barrier()
        pltpu.sync_copy(acc_sh.at[pl.ds(s * RPT, RPT)],
                        out_hbm.at[pl.ds(c * NACC + s * RPT, RPT)])

    return k(vals, idx2d, zinit)


def _lin0(x, w, b):
    def body(x_ref, w_ref, b_ref, o_ref):
        o_ref[...] = jnp.maximum(
            jnp.dot(x_ref[...], w_ref[...], preferred_element_type=F32)
            + b_ref[...], 0.0)

    return pl.pallas_call(
        body, out_shape=jax.ShapeDtypeStruct((N, DIM), F32))(x, w, b)


def _msg(ea_p, p_gath, w1, b1, w2, rsel, fold, b2m):
    nblk = E_PAD // BE
    BF16 = jnp.bfloat16

    def body(ea_ref, p_ref, w1_ref, b1_ref, w2_ref, r_ref, f_ref, b2_ref,
             o_ref):
        eh = jnp.maximum(
            jnp.dot(ea_ref[...], w1_ref[...], preferred_element_type=F32)
            + b1_ref[...], 0.0)
        wedge = jnp.dot(eh.astype(BF16), w2_ref[...],
                        preferred_element_type=F32)
        prep = jnp.dot(p_ref[...].astype(BF16), r_ref[...],
                       preferred_element_type=F32)
        t = (wedge * prep).astype(BF16)
        msg = jnp.dot(t, f_ref[...], preferred_element_type=F32)
        o_ref[...] = msg + jnp.dot(p_ref[...], b2_ref[...],
                                   preferred_element_type=F32)

    return pl.pallas_call(
        body,
        grid=(nblk,),
        in_specs=[
            pl.BlockSpec((BE, 4), lambda i: (i, 0)),
            pl.BlockSpec((BE, DIM), lambda i: (i, 0)),
            pl.BlockSpec((4, 128), lambda i: (0, 0)),
            pl.BlockSpec((1, 128), lambda i: (0, 0)),
            pl.BlockSpec((128, 1024), lambda i: (0, 0)),
            pl.BlockSpec((DIM, 1024), lambda i: (0, 0)),
            pl.BlockSpec((1024, DIM), lambda i: (0, 0)),
            pl.BlockSpec((DIM, DIM), lambda i: (0, 0)),
        ],
        out_specs=pl.BlockSpec((BE, DIM), lambda i: (i, 0)),
        out_shape=jax.ShapeDtypeStruct((E_PAD, DIM), F32),
    )(ea_p, p_gath, w1, b1, w2, rsel, fold, b2m)


def _update(p0, p1, d0, d1, state, cr, cb, wir, wiz, win, whr, whz, whn,
            bir, biz, bin_, bhr, bhz, bhn):
    def body(p0_ref, p1_ref, d0_ref, d1_ref, s_ref, cr_ref, cb_ref,
             wir_ref, wiz_ref, win_ref, whr_ref, whz_ref, whn_ref,
             bir_ref, biz_ref, bin_ref, bhr_ref, bhz_ref, bhn_ref, o_ref):
        h = s_ref[...]
        deg = jnp.maximum(d0_ref[...] + d1_ref[...], 1.0)
        agg = (p0_ref[...] + p1_ref[...]) / deg
        m = jnp.maximum(
            agg + jnp.dot(h, cr_ref[...], preferred_element_type=F32)
            + cb_ref[...], 0.0)
        gir = jnp.dot(m, wir_ref[...], preferred_element_type=F32) + bir_ref[...]
        giz = jnp.dot(m, wiz_ref[...], preferred_element_type=F32) + biz_ref[...]
        gin = jnp.dot(m, win_ref[...], preferred_element_type=F32) + bin_ref[...]
        ghr = jnp.dot(h, whr_ref[...], preferred_element_type=F32) + bhr_ref[...]
        ghz = jnp.dot(h, whz_ref[...], preferred_element_type=F32) + bhz_ref[...]
        ghn = jnp.dot(h, whn_ref[...], preferred_element_type=F32) + bhn_ref[...]
        r = jax.nn.sigmoid(gir + ghr)
        z = jax.nn.sigmoid(giz + ghz)
        n = jnp.tanh(gin + r * ghn)
        o_ref[...] = (1.0 - z) * n + z * h

    return pl.pallas_call(
        body, out_shape=jax.ShapeDtypeStruct((N, DIM), F32))(
            p0, p1, d0, d1, state, cr, cb, wir, wiz, win, whr, whz, whn,
            bir, biz, bin_, bhr, bhz, bhn)


def _set2set(state, bcol, brow, wih, whh, bg, l1w, l1b, l2w, l2b):
    def body(s_ref, bc_ref, br_ref,
             wi0, wi1, wi2, wi3, wh0, wh1, wh2, wh3, b0, b1, b2, b3,
             l1w_ref, l1b_ref, l2w_ref, l2b_ref, o_ref):
        out = s_ref[...]
        oh = (bc_ref[...] == lax.broadcasted_iota(jnp.int32, (N, G), 1)
              ).astype(F32)
        oht = (br_ref[...] == lax.broadcasted_iota(jnp.int32, (G, N), 0)
               ).astype(F32)
        q_star = jnp.zeros((G, 2 * DIM), F32)
        hl = jnp.zeros((G, DIM), F32)
        cl = jnp.zeros((G, DIM), F32)
        wis = (wi0, wi1, wi2, wi3)
        whs = (wh0, wh1, wh2, wh3)
        bs = (b0, b1, b2, b3)
        for _ in range(3):
            gates = [
                jnp.dot(q_star, wis[t][...], preferred_element_type=F32)
                + jnp.dot(hl, whs[t][...], preferred_element_type=F32)
                + bs[t][...]
                for t in range(4)
            ]
            g_i, g_f, g_g, g_o = gates
            cl = jax.nn.sigmoid(g_f) * cl + jax.nn.sigmoid(g_i) * jnp.tanh(g_g)
            hl = jax.nn.sigmoid(g_o) * jnp.tanh(cl)
            q = hl
            qb = jnp.dot(oh, q, preferred_element_type=F32)
            e = jnp.sum(out * qb, axis=1, keepdims=True)
            mm = jnp.where(oh > 0.5, e, -3.0e38)
            emaxt = jnp.max(mm, axis=0, keepdims=True)
            emax_b = jnp.sum(oh * emaxt, axis=1, keepdims=True)
            ee = jnp.exp(e - emax_b)
            denomt = jnp.sum(oh * ee, axis=0, keepdims=True)
            denom_b = jnp.sum(oh * denomt, axis=1, keepdims=True)
            a = ee / denom_b
            rr = jnp.dot(oht, a * out, preferred_element_type=F32)
            q_star = jnp.concatenate([q, rr], axis=1)
        y = jnp.maximum(
            jnp.dot(q_star, l1w_ref[...], preferred_element_type=F32)
            + l1b_ref[...], 0.0)
        o_ref[...] = jnp.dot(y, l2w_ref[...],
                             preferred_element_type=F32) + l2b_ref[...]

    return pl.pallas_call(
        body, out_shape=jax.ShapeDtypeStruct((G, 1), F32))(
            state, bcol, brow, *wih, *whh, *bg, l1w, l1b, l2w, l2b)


def kernel(x, edge_index, edge_attr, batch, do_dropout, mlp_drop,
           lin0_W, lin0_b, enn1_W, enn1_b, enn2_W, enn2_b,
           conv_root, conv_bias, gru_Wih, gru_Whh, gru_bih, gru_bhh,
           s2s_Wih, s2s_Whh, s2s_bih, s2s_bhh, lin1_W, lin1_b, lin2_W, lin2_b):
    del do_dropout, mlp_drop
    pad = E_PAD - E
    src2d = jnp.concatenate(
        [edge_index[0], jnp.zeros((pad,), jnp.int32)]).reshape(IDX_ROWS, 128)
    dst2d = jnp.concatenate(
        [edge_index[1], jnp.full((pad,), N, jnp.int32)]).reshape(IDX_ROWS, 128)
    ea_p = jnp.concatenate(
        [edge_attr, jnp.zeros((pad, 4), F32)], axis=0)
    zinit = jnp.zeros((NACC, DIM), F32)
    ones_e = jnp.ones((E_PAD, DIM), F32)

    rsel = (jnp.arange(DIM * DIM)[None, :] // DIM
            == jnp.arange(DIM)[:, None]).astype(jnp.bfloat16)
    fold = (jnp.arange(DIM * DIM)[:, None] % DIM
            == jnp.arange(DIM)[None, :]).astype(jnp.bfloat16)
    enn2_Wb = enn2_W.astype(jnp.bfloat16)
    b2m = enn2_b.reshape(DIM, DIM)
    b1 = enn1_b.reshape(1, 128)

    wir, wiz, win = (gru_Wih[:, :DIM], gru_Wih[:, DIM:2 * DIM],
                     gru_Wih[:, 2 * DIM:])
    whr, whz, whn = (gru_Whh[:, :DIM], gru_Whh[:, DIM:2 * DIM],
                     gru_Whh[:, 2 * DIM:])
    bir, biz, bin_ = (gru_bih[:DIM].reshape(1, DIM),
                      gru_bih[DIM:2 * DIM].reshape(1, DIM),
                      gru_bih[2 * DIM:].reshape(1, DIM))
    bhr, bhz, bhn = (gru_bhh[:DIM].reshape(1, DIM),
                     gru_bhh[DIM:2 * DIM].reshape(1, DIM),
                     gru_bhh[2 * DIM:].reshape(1, DIM))

    wih = tuple(s2s_Wih[:, t * DIM:(t + 1) * DIM] for t in range(4))
    whh = tuple(s2s_Whh[:, t * DIM:(t + 1) * DIM] for t in range(4))
    bg = tuple((s2s_bih[t * DIM:(t + 1) * DIM]
                + s2s_bhh[t * DIM:(t + 1) * DIM]).reshape(1, DIM)
               for t in range(4))

    state = _lin0(x, lin0_W, lin0_b.reshape(1, DIM))

    degp = _sc_scatter(ones_e, dst2d, zinit)
    d0 = degp[:N, 0:1]
    d1 = degp[NACC:NACC + N, 0:1]

    for _ in range(3):
        p_gath = _sc_gather(state, src2d)
        msg = _msg(ea_p, p_gath, enn1_W, b1, enn2_Wb, rsel, fold, b2m)
        sp = _sc_scatter(msg, dst2d, zinit)
        state = _update(sp[:N], sp[NACC:NACC + N], d0, d1, state,
                        conv_root, conv_bias.reshape(1, DIM),
                        wir, wiz, win, whr, whz, whn,
                        bir, biz, bin_, bhr, bhz, bhn)

    y = _set2set(state, batch.reshape(N, 1), batch.reshape(1, N),
                 wih, whh, bg, lin1_W, lin1_b.reshape(1, DIM),
                 lin2_W, lin2_b.reshape(1, 1))
    return y.reshape(-1)


# speedup vs baseline: 1.1536x; 1.1536x over previous
import functools

import jax
import jax.numpy as jnp
from jax import lax
from jax.experimental import pallas as pl
from jax.experimental.pallas import tpu as pltpu
from jax.experimental.pallas import tpu_sc as plsc

N = 10000
E = 160000
E_PAD = 163840
IDX_ROWS = E_PAD // 128
NUM_FEAT = 128
DIM = 32
G = 64
NACC = 10016
NC, NS = 2, 16
NW = NC * NS
ROWS_PW = IDX_ROWS // NW
RPT = NACC // NS
BE = 640
F32 = jnp.float32

_MESH = plsc.VectorSubcoreMesh(
    core_axis_name="c", subcore_axis_name="s", num_cores=NC, num_subcores=NS)
_SC_PARAMS = pltpu.CompilerParams(use_tc_tiling_on_sc=False)


def _sc_gather(table, idx2d):
    @functools.partial(
        pl.kernel, mesh=_MESH, compiler_params=_SC_PARAMS,
        out_type=jax.ShapeDtypeStruct((E_PAD, DIM), F32),
        scratch_types=[
            pltpu.VMEM((ROWS_PW, 128), jnp.int32),
            pltpu.VMEM((128, DIM), F32),
            pltpu.VMEM((128, DIM), F32),
            pltpu.SemaphoreType.DMA,
            pltpu.SemaphoreType.DMA,
            pltpu.SemaphoreType.DMA,
            pltpu.SemaphoreType.DMA,
        ],
    )
    def k(table_hbm, idx_hbm, out_hbm, idx_v, buf0, buf1, gs0, gs1, ws0, ws1):
        c = lax.axis_index("c")
        s = lax.axis_index("s")
        wid = s * NC + c
        base = wid * ROWS_PW
        pltpu.sync_copy(idx_hbm.at[pl.ds(base, ROWS_PW)], idx_v)

        bufs, gsems, wsems = (buf0, buf1), (gs0, gs1), (ws0, ws1)
        g = [None, None]
        w = [None, None]
        g[0] = pltpu.async_copy(table_hbm.at[idx_v.at[0]], bufs[0], gsems[0])
        for j in range(ROWS_PW):
            b = j & 1
            nb = b ^ 1
            if j + 1 < ROWS_PW:
                if w[nb] is not None:
                    w[nb].wait()
                g[nb] = pltpu.async_copy(
                    table_hbm.at[idx_v.at[j + 1]], bufs[nb], gsems[nb])
            g[b].wait()
            w[b] = pltpu.async_copy(
                bufs[b], out_hbm.at[pl.ds((base + j) * 128, 128)], wsems[b])
        w[0].wait()
        w[1].wait()

    return k(table, idx2d)


def _sc_scatter(vals, idx2d, zinit):
    @functools.partial(
        pl.kernel, mesh=_MESH, compiler_params=_SC_PARAMS,
        out_type=jax.ShapeDtypeStruct((NC * NACC, DIM), F32),
        scratch_types=[
            pltpu.VMEM((ROWS_PW, 128), jnp.int32),
            pltpu.VMEM((128, DIM), F32),
            pltpu.VMEM((128, DIM), F32),
            pltpu.VMEM_SHARED((NACC, DIM), F32),
            pltpu.SemaphoreType.DMA,
            pltpu.SemaphoreType.DMA,
        ],
    )
    def k(val_hbm, idx_hbm, z_hbm, out_hbm, idx_v, buf0, buf1, acc_sh,
          ls0, ls1):
        c = lax.axis_index("c")
        s = lax.axis_index("s")
        wid = s * NC + c
        pltpu.sync_copy(z_hbm.at[pl.ds(s * RPT, RPT)],
                        acc_sh.at[pl.ds(s * RPT, RPT)])
        plsc.subcore_barrier()
        base = wid * ROWS_PW
        pltpu.sync_copy(idx_hbm.at[pl.ds(base, ROWS_PW)], idx_v)

        bufs, lsems = (buf0, buf1), (ls0, ls1)
        l = [None, None]
        l[0] = pltpu.async_copy(
            val_hbm.at[pl.ds(base * 128, 128)], bufs[0], lsems[0])
        for j in range(ROWS_PW):
            b = j & 1
            nb = b ^ 1
            if j + 1 < ROWS_PW:
                l[nb] = pltpu.async_copy(
                    val_hbm.at[pl.ds((base + j + 1) * 128, 128)],
                    bufs[nb], lsems[nb])
            l[b].wait()
            pltpu.sync_copy(bufs[b], acc_sh.at[idx_v.at[j]], add=True)
        plsc.subcore_barrier()
        pltpu.sync_copy(acc_sh.at[pl.ds(s * RPT, RPT)],
                        out_hbm.at[pl.ds(c * NACC + s * RPT, RPT)])

    return k(vals, idx2d, zinit)


def _lin0(x, w, b):
    def body(x_ref, w_ref, b_ref, o_ref):
        o_ref[...] = jnp.maximum(
            jnp.dot(x_ref[...], w_ref[...], preferred_element_type=F32)
            + b_ref[...], 0.0)

    return pl.pallas_call(
        body, out_shape=jax.ShapeDtypeStruct((N, DIM), F32))(x, w, b)


def _msg(ea_p, p_gath, w1, b1, w2, rsel, b2m):
    nblk = E_PAD // BE
    BF16 = jnp.bfloat16

    def body(ea_ref, p_ref, w1_ref, b1_ref, w2_ref, r_ref, b2_ref, o_ref):
        eh = jnp.maximum(
            jnp.dot(ea_ref[...], w1_ref[...], preferred_element_type=F32)
            + b1_ref[...], 0.0)
        wedge = jnp.dot(eh.astype(BF16), w2_ref[...],
                        preferred_element_type=F32)
        prep = jnp.dot(p_ref[...].astype(BF16), r_ref[...],
                       preferred_element_type=F32)
        t = wedge * prep
        acc = ((t[:, 0:128] + t[:, 128:256])
               + (t[:, 256:384] + t[:, 384:512])) + \
              ((t[:, 512:640] + t[:, 640:768])
               + (t[:, 768:896] + t[:, 896:1024]))
        msg = (acc[:, 0:32] + acc[:, 32:64]) + (acc[:, 64:96] + acc[:, 96:128])
        o_ref[...] = msg + jnp.dot(p_ref[...], b2_ref[...],
                                   preferred_element_type=F32)

    return pl.pallas_call(
        body,
        grid=(nblk,),
        in_specs=[
            pl.BlockSpec((BE, 4), lambda i: (i, 0)),
            pl.BlockSpec((BE, DIM), lambda i: (i, 0)),
            pl.BlockSpec((4, 128), lambda i: (0, 0)),
            pl.BlockSpec((1, 128), lambda i: (0, 0)),
            pl.BlockSpec((128, 1024), lambda i: (0, 0)),
            pl.BlockSpec((DIM, 1024), lambda i: (0, 0)),
            pl.BlockSpec((DIM, DIM), lambda i: (0, 0)),
        ],
        out_specs=pl.BlockSpec((BE, DIM), lambda i: (i, 0)),
        out_shape=jax.ShapeDtypeStruct((E_PAD, DIM), F32),
    )(ea_p, p_gath, w1, b1, w2, rsel, b2m)


def _update(p0, p1, d0, d1, state, cr, cb, wir, wiz, win, whr, whz, whn,
            bir, biz, bin_, bhr, bhz, bhn):
    def body(p0_ref, p1_ref, d0_ref, d1_ref, s_ref, cr_ref, cb_ref,
             wir_ref, wiz_ref, win_ref, whr_ref, whz_ref, whn_ref,
             bir_ref, biz_ref, bin_ref, bhr_ref, bhz_ref, bhn_ref, o_ref):
        h = s_ref[...]
        deg = jnp.maximum(d0_ref[...] + d1_ref[...], 1.0)
        agg = (p0_ref[...] + p1_ref[...]) / deg
        m = jnp.maximum(
            agg + jnp.dot(h, cr_ref[...], preferred_element_type=F32)
            + cb_ref[...], 0.0)
        gir = jnp.dot(m, wir_ref[...], preferred_element_type=F32) + bir_ref[...]
        giz = jnp.dot(m, wiz_ref[...], preferred_element_type=F32) + biz_ref[...]
        gin = jnp.dot(m, win_ref[...], preferred_element_type=F32) + bin_ref[...]
        ghr = jnp.dot(h, whr_ref[...], preferred_element_type=F32) + bhr_ref[...]
        ghz = jnp.dot(h, whz_ref[...], preferred_element_type=F32) + bhz_ref[...]
        ghn = jnp.dot(h, whn_ref[...], preferred_element_type=F32) + bhn_ref[...]
        r = jax.nn.sigmoid(gir + ghr)
        z = jax.nn.sigmoid(giz + ghz)
        n = jnp.tanh(gin + r * ghn)
        o_ref[...] = (1.0 - z) * n + z * h

    return pl.pallas_call(
        body, out_shape=jax.ShapeDtypeStruct((N, DIM), F32))(
            p0, p1, d0, d1, state, cr, cb, wir, wiz, win, whr, whz, whn,
            bir, biz, bin_, bhr, bhz, bhn)


def _set2set(state, bcol, brow, wih, whh, bg, l1w, l1b, l2w, l2b):
    def body(s_ref, bc_ref, br_ref,
             wi0, wi1, wi2, wi3, wh0, wh1, wh2, wh3, b0, b1, b2, b3,
             l1w_ref, l1b_ref, l2w_ref, l2b_ref, o_ref):
        out = s_ref[...]
        oh = (bc_ref[...] == lax.broadcasted_iota(jnp.int32, (N, G), 1)
              ).astype(F32)
        oht = (br_ref[...] == lax.broadcasted_iota(jnp.int32, (G, N), 0)
               ).astype(F32)
        q_star = jnp.zeros((G, 2 * DIM), F32)
        hl = jnp.zeros((G, DIM), F32)
        cl = jnp.zeros((G, DIM), F32)
        wis = (wi0, wi1, wi2, wi3)
        whs = (wh0, wh1, wh2, wh3)
        bs = (b0, b1, b2, b3)
        for _ in range(3):
            gates = [
                jnp.dot(q_star, wis[t][...], preferred_element_type=F32)
                + jnp.dot(hl, whs[t][...], preferred_element_type=F32)
                + bs[t][...]
                for t in range(4)
            ]
            g_i, g_f, g_g, g_o = gates
            cl = jax.nn.sigmoid(g_f) * cl + jax.nn.sigmoid(g_i) * jnp.tanh(g_g)
            hl = jax.nn.sigmoid(g_o) * jnp.tanh(cl)
            q = hl
            qb = jnp.dot(oh, q, preferred_element_type=F32)
            e = jnp.sum(out * qb, axis=1, keepdims=True)
            mm = jnp.where(oh > 0.5, e, -3.0e38)
            emaxt = jnp.max(mm, axis=0, keepdims=True)
            emax_b = jnp.sum(oh * emaxt, axis=1, keepdims=True)
            ee = jnp.exp(e - emax_b)
            denomt = jnp.sum(oh * ee, axis=0, keepdims=True)
            denom_b = jnp.sum(oh * denomt, axis=1, keepdims=True)
            a = ee / denom_b
            rr = jnp.dot(oht, a * out, preferred_element_type=F32)
            q_star = jnp.concatenate([q, rr], axis=1)
        y = jnp.maximum(
            jnp.dot(q_star, l1w_ref[...], preferred_element_type=F32)
            + l1b_ref[...], 0.0)
        o_ref[...] = jnp.dot(y, l2w_ref[...],
                             preferred_element_type=F32) + l2b_ref[...]

    return pl.pallas_call(
        body, out_shape=jax.ShapeDtypeStruct((G, 1), F32))(
            state, bcol, brow, *wih, *whh, *bg, l1w, l1b, l2w, l2b)


def kernel(x, edge_index, edge_attr, batch, do_dropout, mlp_drop,
           lin0_W, lin0_b, enn1_W, enn1_b, enn2_W, enn2_b,
           conv_root, conv_bias, gru_Wih, gru_Whh, gru_bih, gru_bhh,
           s2s_Wih, s2s_Whh, s2s_bih, s2s_bhh, lin1_W, lin1_b, lin2_W, lin2_b):
    del do_dropout, mlp_drop
    pad = E_PAD - E
    src2d = jnp.concatenate(
        [edge_index[0], jnp.zeros((pad,), jnp.int32)]).reshape(IDX_ROWS, 128)
    dst2d = jnp.concatenate(
        [edge_index[1], jnp.full((pad,), N, jnp.int32)]).reshape(IDX_ROWS, 128)
    ea_p = jnp.concatenate(
        [edge_attr, jnp.zeros((pad, 4), F32)], axis=0)
    zinit = jnp.zeros((NACC, DIM), F32)
    ones_e = jnp.ones((E_PAD, DIM), F32)

    rsel = (jnp.arange(DIM * DIM)[None, :] // DIM
            == jnp.arange(DIM)[:, None]).astype(jnp.bfloat16)
    enn2_Wb = enn2_W.astype(jnp.bfloat16)
    b2m = enn2_b.reshape(DIM, DIM)
    b1 = enn1_b.reshape(1, 128)

    wir, wiz, win = (gru_Wih[:, :DIM], gru_Wih[:, DIM:2 * DIM],
                     gru_Wih[:, 2 * DIM:])
    whr, whz, whn = (gru_Whh[:, :DIM], gru_Whh[:, DIM:2 * DIM],
                     gru_Whh[:, 2 * DIM:])
    bir, biz, bin_ = (gru_bih[:DIM].reshape(1, DIM),
                      gru_bih[DIM:2 * DIM].reshape(1, DIM),
                      gru_bih[2 * DIM:].reshape(1, DIM))
    bhr, bhz, bhn = (gru_bhh[:DIM].reshape(1, DIM),
                     gru_bhh[DIM:2 * DIM].reshape(1, DIM),
                     gru_bhh[2 * DIM:].reshape(1, DIM))

    wih = tuple(s2s_Wih[:, t * DIM:(t + 1) * DIM] for t in range(4))
    whh = tuple(s2s_Whh[:, t * DIM:(t + 1) * DIM] for t in range(4))
    bg = tuple((s2s_bih[t * DIM:(t + 1) * DIM]
                + s2s_bhh[t * DIM:(t + 1) * DIM]).reshape(1, DIM)
               for t in range(4))

    state = _lin0(x, lin0_W, lin0_b.reshape(1, DIM))

    degp = _sc_scatter(ones_e, dst2d, zinit)
    d0 = degp[:N, 0:1]
    d1 = degp[NACC:NACC + N, 0:1]

    for _ in range(3):
        p_gath = _sc_gather(state, src2d)
        msg = _msg(ea_p, p_gath, enn1_W, b1, enn2_Wb, rsel, b2m)
        sp = _sc_scatter(msg, dst2d, zinit)
        state = _update(sp[:N], sp[NACC:NACC + N], d0, d1, state,
                        conv_root, conv_bias.reshape(1, DIM),
                        wir, wiz, win, whr, whz, whn,
                        bir, biz, bin_, bhr, bhz, bhn)

    y = _set2set(state, batch.reshape(N, 1), batch.reshape(1, N),
                 wih, whh, bg, lin1_W, lin1_b.reshape(1, DIM),
                 lin2_W, lin2_b.reshape(1, 1))
    return y.reshape(-1)
